# baseline (speedup 1.0000x reference)
import jax
import jax.numpy as jnp
from jax import lax
from jax.experimental import pallas as pl
from jax.experimental.pallas import tpu as pltpu
from jax.experimental.pallas import tpu_sc as plsc

N1 = 4096
N2 = 16384
K = 128
NC = 2
NS = 16
L = 16
NW = NC * NS
ROWS_PER_W = N1 // NW
NSEG = 128
SEG_G = 8
SEG_V = N2 // NSEG
BIG_NEG = -3.0e38
NR = 3
NBUF = 2 * NR
NBODY = 21


def _scalar(x):
    if getattr(x, "shape", ()) == (L,):
        return x[0]
    return x


def _maxtree(vs):
    while len(vs) > 1:
        vs = [jnp.maximum(vs[2 * i], vs[2 * i + 1]) for i in range(len(vs) // 2)] + (
            [vs[-1]] if len(vs) % 2 else []
        )
    return vs[0]


def _mintree(vs):
    while len(vs) > 1:
        vs = [jnp.minimum(vs[2 * i], vs[2 * i + 1]) for i in range(len(vs) // 2)] + (
            [vs[-1]] if len(vs) % 2 else []
        )
    return vs[0]


def _body(neural_hbm, feats_hbm, wb_hbm, out_hbm,
          rb0, rb1, rb2, rb3, rb4, rb5, fb0, fb1, fb2, fb3, fb4, fb5,
          ib0, ib1, ib2, wbbuf, sem_in, sem_fin, sem_out):
    rowbufs = [rb0, rb1, rb2, rb3, rb4, rb5]
    featbufs = [fb0, fb1, fb2, fb3, fb4, fb5]
    idxbufs = [ib0, ib1, ib2]
    wid = lax.axis_index("s") * NC + lax.axis_index("c")
    base_row = wid * ROWS_PER_W

    pltpu.sync_copy(wb_hbm, wbbuf)
    wv = wbbuf[pl.ds(0, L)]
    w0, w1, w2, b0 = wv[0], wv[1], wv[2], wv[3]

    iota = lax.iota(jnp.int32, L)
    bases = [iota * NSEG + (L * NSEG) * t for t in range(SEG_G)]
    colvecs = [iota + L * g for g in range(SEG_G)]

    def issue_in(b, row):
        return (
            pltpu.async_copy(neural_hbm.at[row], rowbufs[b], sem_in.at[b]),
            pltpu.async_copy(feats_hbm.at[row], featbufs[b], sem_fin.at[b]),
        )

    def wait_in(b, row):
        pltpu.make_async_copy(neural_hbm.at[row], rowbufs[b],
                              sem_in.at[b]).wait()
        pltpu.make_async_copy(feats_hbm.at[row], featbufs[b],
                              sem_fin.at[b]).wait()

    def issue_out(b, row):
        return pltpu.async_copy(rowbufs[b], out_hbm.at[row], sem_out.at[b])

    def wait_out(b, row):
        pltpu.make_async_copy(rowbufs[b], out_hbm.at[row],
                              sem_out.at[b]).wait()

    def compute_pair(bufs, fbufs, ibufs, rows):
        NR = len(bufs)

        def seg_step(v4, MV):
            Ms, Vs = MV
            off = v4 * (NSEG * 4)
            for u in range(4):
                eoff = off + u * NSEG
                newM, newV = [], []
                for s in range(NR):
                    ms, vs = [], []
                    for g in range(SEG_G):
                        x = bufs[s][pl.ds(eoff + g * L, L)]
                        m2 = jnp.maximum(Ms[s][g], x)
                        vs.append(jnp.where(m2 != Ms[s][g],
                                            colvecs[g] + eoff, Vs[s][g]))
                        ms.append(m2)
                    newM.append(tuple(ms))
                    newV.append(tuple(vs))
                Ms, Vs = tuple(newM), tuple(newV)
            return Ms, Vs
        M, V = lax.fori_loop(
            0, SEG_V // 4, seg_step,
            (tuple(tuple(jnp.full((L,), BIG_NEG, jnp.float32)
                         for _ in range(SEG_G)) for _ in range(NR)),
             tuple(tuple(jnp.zeros((L,), jnp.int32)
                         for _ in range(SEG_G)) for _ in range(NR))),
        )

        lane0 = iota == 0

        def min_pos(seg, val):
            vnew = jnp.int32(99999)
            for t in range(SEG_G):
                mt = seg[t] == val
                pc = _scalar(plsc.all_reduce_population_count(mt))
                ff = _scalar(plsc.all_reduce_ffs(mt))
                vnew = jnp.minimum(
                    vnew, jnp.where(pc > 0, ff + L * t, jnp.int32(99999)))
            return vnew

        def one_extract(buf, M, V, gmax):
            ecand = [jnp.where(M[g] == gmax, V[g], jnp.int32(0x7FFFFFF))
                     for g in range(SEG_G)]
            e = _scalar(jnp.min(_mintree(ecand)))
            col = jnp.remainder(e, NSEG)
            upd = [colvecs[g] == col for g in range(SEG_G)]
            plsc.store_scatter(buf, [jnp.full((L,), e, jnp.int32)],
                               jnp.full((L,), BIG_NEG, jnp.float32),
                               mask=lane0)
            seg = [plsc.load_gather(buf, [bases[t] + col])
                   for t in range(SEG_G)]
            newmax = _scalar(jnp.max(_maxtree(list(seg))))
            ev = min_pos(seg, newmax) * NSEG + col
            M2 = tuple(jnp.where(upd[g], newmax, M[g]) for g in range(SEG_G))
            V2 = tuple(jnp.where(upd[g], ev, V[g]) for g in range(SEG_G))
            return M2, V2, e

        def extract(k, carry):
            M, V, csum, gmaxs = carry
            kvec1 = jnp.full((L,), 0, jnp.int32) + 2 * k
            kvec2 = kvec1 + 1

            es1, cols1, upds1 = [], [], []
            m2s, es2, cols2, upds2, Mxs = [], [], [], [], []
            for s in range(NR):
                ecand = [jnp.where(M[s][g] == gmaxs[s], V[s][g],
                                   jnp.int32(0x7FFFFFF))
                         for g in range(SEG_G)]
                e1 = _scalar(jnp.min(_mintree(ecand)))
                col1 = jnp.remainder(e1, NSEG)
                upd1 = [colvecs[g] == col1 for g in range(SEG_G)]
                Mx = [jnp.where(upd1[g], jnp.float32(BIG_NEG), M[s][g])
                      for g in range(SEG_G)]
                m2 = _scalar(jnp.max(_maxtree(list(Mx))))
                ecand2 = [jnp.where(Mx[g] == m2, V[s][g],
                                    jnp.int32(0x7FFFFFF))
                          for g in range(SEG_G)]
                e2 = _scalar(jnp.min(_mintree(ecand2)))
                col2 = jnp.remainder(e2, NSEG)
                es1.append(e1); cols1.append(col1); upds1.append(upd1)
                m2s.append(m2); es2.append(e2); cols2.append(col2)
                upds2.append([colvecs[g] == col2 for g in range(SEG_G)])
                Mxs.append(Mx)

            for s in range(NR):
                plsc.store_scatter(ibufs[s], [kvec1],
                                   jnp.full((L,), es1[s], jnp.int32),
                                   mask=lane0)
            for s in range(NR):
                plsc.store_scatter(bufs[s],
                                   [jnp.full((L,), es1[s], jnp.int32)],
                                   jnp.full((L,), BIG_NEG, jnp.float32),
                                   mask=lane0)
                plsc.store_scatter(bufs[s],
                                   [jnp.full((L,), es2[s], jnp.int32)],
                                   jnp.full((L,), BIG_NEG, jnp.float32),
                                   mask=lane0)
            out_M, out_V, out_csum, out_gmax = [], [], [], []
            for s in range(NR):
                seg1 = [plsc.load_gather(bufs[s], [bases[t] + cols1[s]])
                        for t in range(SEG_G)]
                seg2 = [plsc.load_gather(bufs[s], [bases[t] + cols2[s]])
                        for t in range(SEG_G)]
                newmax1 = _scalar(jnp.max(_maxtree(list(seg1))))
                ev1 = min_pos(seg1, newmax1) * NSEG + cols1[s]
                newmax2 = _scalar(jnp.max(_maxtree(list(seg2))))
                ev2 = min_pos(seg2, newmax2) * NSEG + cols2[s]

                def common(_, s=s, newmax1=newmax1, ev1=ev1,
                           newmax2=newmax2, ev2=ev2):
                    Mc = tuple(
                        jnp.where(upds1[s][g], newmax1,
                                  jnp.where(upds2[s][g], newmax2, M[s][g]))
                        for g in range(SEG_G))
                    Vc = tuple(
                        jnp.where(upds1[s][g], ev1,
                                  jnp.where(upds2[s][g], ev2, V[s][g]))
                        for g in range(SEG_G))
                    m3 = _scalar(jnp.max(_maxtree([
                        jnp.where(upds2[s][g], jnp.float32(BIG_NEG), Mxs[s][g])
                        for g in range(SEG_G)])))
                    gnext = jnp.maximum(jnp.maximum(m3, newmax1), newmax2)
                    return Mc + Vc + (m2s[s], gnext, es2[s])

                def rare(_, s=s, newmax1=newmax1, ev1=ev1):
                    plsc.store_scatter(
                        bufs[s], [jnp.full((L,), es2[s], jnp.int32)],
                        jnp.broadcast_to(m2s[s], (L,)), mask=lane0)
                    M1 = tuple(jnp.where(upds1[s][g], newmax1, M[s][g])
                               for g in range(SEG_G))
                    V1 = tuple(jnp.where(upds1[s][g], ev1, V[s][g])
                               for g in range(SEG_G))
                    g1 = jnp.maximum(m2s[s], newmax1)
                    M2_, V2_, e2r = one_extract(bufs[s], M1, V1, g1)
                    gnext = _scalar(jnp.max(_maxtree(list(M2_))))
                    return M2_ + V2_ + (g1, gnext, e2r)

                res = lax.cond(newmax1 < m2s[s], common, rare, 0)
                Mn = tuple(res[0:SEG_G])
                Vn = tuple(res[SEG_G:2 * SEG_G])
                val2, gnext, e2f = res[2 * SEG_G:]
                plsc.store_scatter(ibufs[s], [kvec2],
                                   jnp.full((L,), e2f, jnp.int32),
                                   mask=lane0)
                out_M.append(Mn)
                out_V.append(Vn)
                out_csum.append(csum[s] + gmaxs[s] + val2)
                out_gmax.append(gnext)
            return (tuple(out_M), tuple(out_V), tuple(out_csum),
                    tuple(out_gmax))

        gmax0 = tuple(_scalar(jnp.max(_maxtree(list(M[s]))))
                      for s in range(NR))
        M, V, csum, _ = lax.fori_loop(
            0, K // 2, extract,
            (M, V, tuple(jnp.float32(0.0) for _ in range(NR)), gmax0))

        for s in range(NR):
            ssum = jnp.zeros((L,), jnp.float32)
            srcs = []
            for j in range(K // L):
                fbase = (iota + j * L) * 3
                f0 = plsc.load_gather(fbufs[s], [fbase])
                f1 = plsc.load_gather(fbufs[s], [fbase + 1])
                f2 = plsc.load_gather(fbufs[s], [fbase + 2])
                sc = jnp.exp(f0 * w0 + f1 * w1 + f2 * w2 + b0)
                srcs.append(sc)
                ssum = ssum + sc
            l1 = jnp.maximum(_scalar(jnp.sum(ssum)), jnp.float32(1e-12))
            scale = jnp.broadcast_to(csum[s], (L,)) / jnp.broadcast_to(l1, (L,))
            for j in range(K // L):
                idx = ibufs[s][pl.ds(j * L, L)]
                plsc.store_scatter(bufs[s], [idx], srcs[j] * scale)

    for b in range(NR):
        issue_in(b, base_row + b)

    def pipeline_body(i2, _):
        q = base_row + i2 * NBUF

        @pl.when(i2 > 0)
        def _():
            for b in range(NR):
                wait_out(NR + b, q - NR + b)

        for b in range(NR):
            issue_in(NR + b, q + NR + b)

        for b in range(NR):
            wait_in(b, q + b)
        compute_pair([rowbufs[b] for b in range(NR)],
                     [featbufs[b] for b in range(NR)], idxbufs,
                     tuple(q + b for b in range(NR)))
        for b in range(NR):
            issue_out(b, q + b)

        for b in range(NR):
            wait_in(NR + b, q + NR + b)
        compute_pair([rowbufs[NR + b] for b in range(NR)],
                     [featbufs[NR + b] for b in range(NR)], idxbufs,
                     tuple(q + NR + b for b in range(NR)))
        for b in range(NR):
            issue_out(NR + b, q + NR + b)

        for b in range(NR):
            wait_out(b, q + b)

        @pl.when(i2 < NBODY - 1)
        def _():
            for b in range(NR):
                issue_in(b, q + NBUF + b)

        return 0

    lax.fori_loop(0, NBODY, pipeline_body, 0)
    last = base_row + (NBODY - 1) * NBUF
    for b in range(NR):
        wait_out(NR + b, last + NR + b)

    tail = base_row + NBODY * NBUF
    issue_in(0, tail)
    issue_in(1, tail + 1)
    wait_in(0, tail)
    wait_in(1, tail + 1)
    compute_pair([rowbufs[0], rowbufs[1]], [featbufs[0], featbufs[1]],
                 idxbufs, (tail, tail + 1))
    issue_out(0, tail)
    issue_out(1, tail + 1)
    wait_out(0, tail)
    wait_out(1, tail + 1)


@jax.jit
def kernel(neural_prob_mtx, features, W, b):
    feats = features.reshape(N1, K * 3)
    wb = jnp.zeros((16,), jnp.float32)
    wb = wb.at[0].set(W[0, 0]).at[1].set(W[1, 0]).at[2].set(W[2, 0]).at[3].set(b[0])

    mesh = plsc.VectorSubcoreMesh(core_axis_name="c", subcore_axis_name="s")
    run = pl.kernel(
        _body,
        out_type=jax.ShapeDtypeStruct((N1, N2), jnp.float32),
        mesh=mesh,
        scratch_types=[pltpu.VMEM((N2,), jnp.float32)] * NBUF
          + [pltpu.VMEM((K * 3,), jnp.float32)] * NBUF
          + [pltpu.VMEM((K,), jnp.int32)] * NR
          + [
            pltpu.VMEM((16,), jnp.float32),
            pltpu.SemaphoreType.DMA((NBUF,)),
            pltpu.SemaphoreType.DMA((NBUF,)),
            pltpu.SemaphoreType.DMA((NBUF,)),
        ],
        compiler_params=pltpu.CompilerParams(needs_layout_passes=False),
    )
    return run(neural_prob_mtx, feats, wb)
